# Initial kernel scaffold; baseline (speedup 1.0000x reference)
import jax
import jax.numpy as jnp
from jax.experimental import pallas as pl


def kernel(x_customer, x_product, edge_index_cbp, edge_index_pbc, W1_cbp_l, b1_cbp, W1_cbp_r, W1_pbc_l, b1_pbc, W1_pbc_r, W2_cbp_l, b2_cbp, W2_cbp_r, W2_pbc_l, b2_pbc, W2_pbc_r, W_lin, b_lin):
    raise NotImplementedError("write your pallas kernel here")


# speedup vs baseline: 4.9232x; 4.9232x over previous
import functools

import jax
import jax.numpy as jnp
from jax import lax
from jax.experimental import pallas as pl
from jax.experimental.pallas import tpu as pltpu
from jax.experimental.pallas import tpu_sc as plsc

N = 10000
NPAD = 10240
ROWS_PER_TILE = NPAD // 16
E = 320000
CH = 64
EROWS = 5120
G = 32
NG_A = 10
NG_C = 5
D = 128
DO = 64

_mesh = plsc.VectorSubcoreMesh(core_axis_name="c", subcore_axis_name="s")


def _edge_pipeline(tab, s2d, d2d, row0, ngroups, idx_s, idx_d, rows0, rows1,
                   sem0, sem1, acc_sh, onesv, cnt_sh):
    def group(gi, carry):
        grow = row0 + gi * G
        pltpu.sync_copy(s2d.at[pl.ds(grow, G)], idx_s)
        pltpu.sync_copy(d2d.at[pl.ds(grow, G)], idx_d)

        pltpu.async_copy(tab.at[idx_s.at[0]], rows0, sem0)
        pltpu.async_copy(tab.at[idx_s.at[1]], rows1, sem1)

        def body(j, c):
            i0 = 2 * j

            pltpu.make_async_copy(tab.at[idx_s.at[i0]], rows0, sem0).wait()
            pltpu.sync_copy(rows0, acc_sh.at[idx_d.at[i0]], add=True)
            if cnt_sh is not None:
                pltpu.sync_copy(onesv, cnt_sh.at[idx_d.at[i0]], add=True)

            @pl.when(j + 1 < G // 2)
            def _():
                pltpu.async_copy(tab.at[idx_s.at[i0 + 2]], rows0, sem0)

            pltpu.make_async_copy(tab.at[idx_s.at[i0 + 1]], rows1, sem1).wait()
            pltpu.sync_copy(rows1, acc_sh.at[idx_d.at[i0 + 1]], add=True)
            if cnt_sh is not None:
                pltpu.sync_copy(onesv, cnt_sh.at[idx_d.at[i0 + 1]], add=True)

            @pl.when(j + 1 < G // 2)
            def _():
                pltpu.async_copy(tab.at[idx_s.at[i0 + 3]], rows1, sem1)

            return c

        return lax.fori_loop(0, G // 2, body, carry)

    lax.fori_loop(0, ngroups, group, 0)


def _stage_a_body(xc, xp, s_all, d_all, z128, z16, ones_h,
                  aggp, cntp, aggc, cntc,
                  idx_s, idx_d, rows0, rows1, onesv, acc_sh, cnt_sh,
                  sem0, sem1):
    cid = lax.axis_index("c")
    sid = lax.axis_index("s")
    row0 = sid * ROWS_PER_TILE

    pltpu.sync_copy(z16, onesv)
    pltpu.sync_copy(z128, rows0)
    for t in range(ROWS_PER_TILE // CH):
        r = row0 + t * CH
        pltpu.sync_copy(rows0, acc_sh.at[pl.ds(r, CH)])
        pltpu.sync_copy(onesv, cnt_sh.at[pl.ds(r, CH)])
    pltpu.sync_copy(ones_h, onesv)
    plsc.subcore_barrier()

    erow0 = sid * (NG_A * G)

    @pl.when(cid == 0)
    def _():
        _edge_pipeline(xc, s_all.at[0], d_all.at[0], erow0, NG_A,
                       idx_s, idx_d, rows0, rows1, sem0, sem1,
                       acc_sh, onesv, cnt_sh)

    @pl.when(cid == 1)
    def _():
        _edge_pipeline(xp, s_all.at[1], d_all.at[1], erow0, NG_A,
                       idx_s, idx_d, rows0, rows1, sem0, sem1,
                       acc_sh, onesv, cnt_sh)

    plsc.subcore_barrier()

    def _writeback(agg_out, cnt_out):
        for t in range(ROWS_PER_TILE // CH):
            r = row0 + t * CH
            pltpu.sync_copy(acc_sh.at[pl.ds(r, CH)], rows0)
            pltpu.sync_copy(rows0, agg_out.at[pl.ds(r, CH)])
            pltpu.sync_copy(cnt_sh.at[pl.ds(r, CH)], onesv)
            pltpu.sync_copy(onesv, cnt_out.at[pl.ds(r, CH)])

    @pl.when(cid == 0)
    def _():
        _writeback(aggp, cntp)

    @pl.when(cid == 1)
    def _():
        _writeback(aggc, cntc)


_stage_a = functools.partial(
    pl.kernel,
    out_type=[
        jax.ShapeDtypeStruct((NPAD, D), jnp.float32),
        jax.ShapeDtypeStruct((NPAD, 16), jnp.float32),
        jax.ShapeDtypeStruct((NPAD, D), jnp.float32),
        jax.ShapeDtypeStruct((NPAD, 16), jnp.float32),
    ],
    mesh=_mesh,
    scratch_types=[
        pltpu.VMEM((G, CH), jnp.int32),
        pltpu.VMEM((G, CH), jnp.int32),
        pltpu.VMEM((CH, D), jnp.float32),
        pltpu.VMEM((CH, D), jnp.float32),
        pltpu.VMEM((CH, 16), jnp.float32),
        pltpu.VMEM_SHARED((NPAD, D), jnp.float32),
        pltpu.VMEM_SHARED((NPAD, 16), jnp.float32),
        pltpu.SemaphoreType.DMA,
        pltpu.SemaphoreType.DMA,
    ],
    compiler_params=pltpu.CompilerParams(use_tc_tiling_on_sc=False),
)(_stage_a_body)


def _stage_c_body(g, s_all, d_all, z64,
                  agg2,
                  idx_s, idx_d, rows0, rows1, acc_sh, sem0, sem1):
    cid = lax.axis_index("c")
    sid = lax.axis_index("s")
    row0 = sid * ROWS_PER_TILE

    pltpu.sync_copy(z64, rows0)
    for t in range(ROWS_PER_TILE // CH):
        pltpu.sync_copy(rows0, acc_sh.at[pl.ds(row0 + t * CH, CH)])
    plsc.subcore_barrier()

    wid = sid * 2 + cid
    erow0 = wid * (NG_C * G)
    _edge_pipeline(g, s_all.at[1], d_all.at[1], erow0, NG_C,
                   idx_s, idx_d, rows0, rows1, sem0, sem1,
                   acc_sh, None, None)

    plsc.subcore_barrier()

    def _writeback(out2d):
        for t in range(ROWS_PER_TILE // CH):
            r = row0 + t * CH
            pltpu.sync_copy(acc_sh.at[pl.ds(r, CH)], rows0)
            pltpu.sync_copy(rows0, out2d.at[pl.ds(r, CH)])

    @pl.when(cid == 0)
    def _():
        _writeback(agg2.at[0])

    @pl.when(cid == 1)
    def _():
        _writeback(agg2.at[1])


_stage_c = functools.partial(
    pl.kernel,
    out_type=[jax.ShapeDtypeStruct((2, NPAD, DO), jnp.float32)],
    mesh=_mesh,
    scratch_types=[
        pltpu.VMEM((G, CH), jnp.int32),
        pltpu.VMEM((G, CH), jnp.int32),
        pltpu.VMEM((CH, DO), jnp.float32),
        pltpu.VMEM((CH, DO), jnp.float32),
        pltpu.VMEM_SHARED((NPAD, DO), jnp.float32),
        pltpu.SemaphoreType.DMA,
        pltpu.SemaphoreType.DMA,
    ],
    compiler_params=pltpu.CompilerParams(use_tc_tiling_on_sc=False),
)(_stage_c_body)


_BLK = 1000


def _dot(a, b):
    return jnp.dot(a, b, preferred_element_type=jnp.float32,
                   precision=lax.Precision.HIGHEST)


def _stage_b_kern(aggp, cntp, xp, aggc, cntc, xc,
                  W1cl, b1c, W1cr, W1pl, b1p, W1pr,
                  W2pl, W2pr, WL, b2p, bL,
                  g_out, z_out):
    mean_p = aggp[...] / jnp.maximum(cntp[:, 0:1], 1.0)
    h_p = jnp.maximum(
        _dot(mean_p, W1cl[...]) + b1c[...] + _dot(xp[...], W1cr[...]), 0.0)
    g_out[...] = _dot(h_p, _dot(W2pl[...], WL[...]))

    mean_c = aggc[...] / jnp.maximum(cntc[:, 0:1], 1.0)
    h_c = jnp.maximum(
        _dot(mean_c, W1pl[...]) + b1p[...] + _dot(xc[...], W1pr[...]), 0.0)
    z_out[...] = (_dot(h_c, _dot(W2pr[...], WL[...]))
                  + _dot(b2p[...], WL[...]) + bL[...])


def _stage_b(aggp, cntp, xp, aggc, cntc, xc,
             W1cl, b1c, W1cr, W1pl, b1p, W1pr, W2pl, W2pr, WL, b2p, bL):
    row_spec = lambda w: pl.BlockSpec((_BLK, w), lambda i: (i, 0))
    full = lambda a: pl.BlockSpec(a.shape, lambda i: (0,) * a.ndim)
    return pl.pallas_call(
        _stage_b_kern,
        grid=(N // _BLK,),
        in_specs=[
            row_spec(D), row_spec(16), row_spec(D),
            row_spec(D), row_spec(16), row_spec(D),
            full(W1cl), full(b1c), full(W1cr),
            full(W1pl), full(b1p), full(W1pr),
            full(W2pl), full(W2pr), full(WL), full(b2p), full(bL),
        ],
        out_specs=[row_spec(DO), row_spec(DO)],
        out_shape=[
            jax.ShapeDtypeStruct((N, DO), jnp.float32),
            jax.ShapeDtypeStruct((N, DO), jnp.float32),
        ],
    )(aggp, cntp, xp, aggc, cntc, xc,
      W1cl, b1c, W1cr, W1pl, b1p, W1pr, W2pl, W2pr, WL, b2p, bL)


def _stage_d_kern(p0, p1, cntc, z, out):
    out[...] = ((p0[...] + p1[...]) / jnp.maximum(cntc[:, 0:1], 1.0)
                + z[...])


def _stage_d(p0, p1, cntc, z):
    row_spec = lambda w: pl.BlockSpec((_BLK, w), lambda i: (i, 0))
    return pl.pallas_call(
        _stage_d_kern,
        grid=(N // _BLK,),
        in_specs=[row_spec(DO), row_spec(DO), row_spec(16), row_spec(DO)],
        out_specs=row_spec(DO),
        out_shape=jax.ShapeDtypeStruct((N, DO), jnp.float32),
    )(p0, p1, cntc, z)


def _pad_edges(ei):
    src = ei[0].astype(jnp.int32)
    dst = ei[1].astype(jnp.int32)
    pad = EROWS * CH - E
    src = jnp.concatenate([src, jnp.zeros((pad,), jnp.int32)])
    dst = jnp.concatenate([dst, jnp.full((pad,), N, jnp.int32)])
    return src.reshape(EROWS, CH), dst.reshape(EROWS, CH)


def kernel(x_customer, x_product, edge_index_cbp, edge_index_pbc,
           W1_cbp_l, b1_cbp, W1_cbp_r, W1_pbc_l, b1_pbc, W1_pbc_r,
           W2_cbp_l, b2_cbp, W2_cbp_r, W2_pbc_l, b2_pbc, W2_pbc_r,
           W_lin, b_lin):
    s_cbp, d_cbp = _pad_edges(edge_index_cbp)
    s_pbc, d_pbc = _pad_edges(edge_index_pbc)
    s_all = jnp.stack([s_cbp, s_pbc])
    d_all = jnp.stack([d_cbp, d_pbc])

    z128 = jnp.zeros((CH, D), jnp.float32)
    z16 = jnp.zeros((CH, 16), jnp.float32)
    z64 = jnp.zeros((CH, DO), jnp.float32)
    ones_h = jnp.ones((CH, 16), jnp.float32)

    aggp, cntp, aggc, cntc = _stage_a(
        x_customer, x_product, s_all, d_all, z128, z16, ones_h)

    g, z = _stage_b(
        aggp[:N], cntp[:N], x_product, aggc[:N], cntc[:N], x_customer,
        W1_cbp_l, b1_cbp.reshape(1, D), W1_cbp_r,
        W1_pbc_l, b1_pbc.reshape(1, D), W1_pbc_r,
        W2_pbc_l, W2_pbc_r, W_lin, b2_pbc.reshape(1, D),
        b_lin.reshape(1, DO))

    (agg2,) = _stage_c(g, s_all, d_all, z64)

    return _stage_d(agg2[0, :N], agg2[1, :N], cntc[:N], z)
